# Initial kernel scaffold; baseline (speedup 1.0000x reference)
import jax
import jax.numpy as jnp
from jax.experimental import pallas as pl


def kernel(x, emb_tables, lin_w, lin_b, w1, b1, w2, b2):
    raise NotImplementedError("write your pallas kernel here")


# speedup vs baseline: 7.4096x; 7.4096x over previous
import functools

import jax
import jax.numpy as jnp
from jax import lax
from jax.experimental import pallas as pl
from jax.experimental.pallas import tpu as pltpu
from jax.experimental.pallas import tpu_sc as plsc

B = 16384
F = 26
V = 100000
D = 16
ADAPT = 128

NC = 2
NS = 16
NW = NC * NS
ROWS = B * F
RPW = ROWS // NW
CHUNK = 128
NCH = RPW // CHUNK


def _gather_body(table_hbm, idx_hbm, out_hbm, idx_v, buf, sem):
    wid = lax.axis_index("s") * NC + lax.axis_index("c")
    base = wid * RPW
    pltpu.sync_copy(idx_hbm.at[wid], idx_v)

    def body(j, carry):
        pltpu.async_copy(table_hbm.at[idx_v.at[j]], buf, sem).wait()
        pltpu.sync_copy(buf, out_hbm.at[pl.ds(base + j * CHUNK, CHUNK)])
        return carry

    lax.fori_loop(0, NCH, body, 0)


@functools.partial(jax.jit, static_argnums=())
def _sc_gather(table, idx3):
    mesh = plsc.VectorSubcoreMesh(core_axis_name="c", subcore_axis_name="s")
    return pl.kernel(
        _gather_body,
        out_type=jax.ShapeDtypeStruct((ROWS, D), jnp.float32),
        mesh=mesh,
        scratch_types=[
            pltpu.VMEM((NCH, CHUNK), jnp.int32),
            pltpu.VMEM((CHUNK, D), jnp.float32),
            pltpu.SemaphoreType.DMA,
        ],
        compiler_params=pltpu.CompilerParams(use_tc_tiling_on_sc=False),
    )(table, idx3)


def _dense_body(feat_ref, lw_ref, lb_ref, w1_ref, b1_ref, w2_ref, b2_ref,
                out_ref):
    h = jnp.dot(feat_ref[...], lw_ref[...],
                preferred_element_type=jnp.float32) + lb_ref[...]
    h = jnp.maximum(jnp.dot(h, w1_ref[...],
                            preferred_element_type=jnp.float32) + b1_ref[...],
                    0.0)
    out_ref[...] = jnp.dot(h, w2_ref[...],
                           preferred_element_type=jnp.float32) + b2_ref[...]


BLK = 2048


def _tc_dense(feat, lin_w, lin_b, w1, b1, w2, b2):
    grid = (B // BLK,)
    return pl.pallas_call(
        _dense_body,
        grid=grid,
        in_specs=[
            pl.BlockSpec((BLK, F * D), lambda i: (i, 0)),
            pl.BlockSpec((F * D, ADAPT), lambda i: (0, 0)),
            pl.BlockSpec((1, ADAPT), lambda i: (0, 0)),
            pl.BlockSpec((ADAPT, ADAPT), lambda i: (0, 0)),
            pl.BlockSpec((1, ADAPT), lambda i: (0, 0)),
            pl.BlockSpec((ADAPT, 1), lambda i: (0, 0)),
            pl.BlockSpec((1, 1), lambda i: (0, 0)),
        ],
        out_specs=pl.BlockSpec((BLK, 1), lambda i: (i, 0)),
        out_shape=jax.ShapeDtypeStruct((B, 1), jnp.float32),
    )(feat, lin_w, lin_b, w1, b1, w2, b2)


def kernel(x, emb_tables, lin_w, lin_b, w1, b1, w2, b2):
    table = emb_tables.reshape(F * V, D)
    flat_idx = (x + (jnp.arange(F, dtype=jnp.int32) * V)[None, :]).reshape(-1)
    idx3 = flat_idx.reshape(NW, NCH, CHUNK)
    rows = _sc_gather(table, idx3)
    feat = rows.reshape(B, F * D)
    return _tc_dense(feat, lin_w, lin_b.reshape(1, ADAPT), w1,
                     b1.reshape(1, ADAPT), w2, b2.reshape(1, 1))


# speedup vs baseline: 18.3263x; 2.4733x over previous
import functools

import jax
import jax.numpy as jnp
from jax import lax
from jax.experimental import pallas as pl
from jax.experimental.pallas import tpu as pltpu
from jax.experimental.pallas import tpu_sc as plsc

B = 16384
F = 26
V = 100000
D = 16
ADAPT = 128

NC = 2
NS = 16
NW = NC * NS
K = F * D
RPW = K // NW
BH = B // 2


def _rowsweep_body(tab_hbm, xt_hbm, out_hbm, row_v, x_v, o_v):
    wid = lax.axis_index("s") * NC + lax.axis_index("c")

    def row_step(j, carry):
        k = wid * RPW + j
        f = k // D
        d = k % D
        pltpu.sync_copy(tab_hbm.at[f, d], row_v)

        def half_step(h, carry2):
            b0 = h * BH
            pltpu.sync_copy(xt_hbm.at[f, pl.ds(b0, BH)], x_v)

            def gather_step(i, carry3):
                idx = x_v[pl.ds(i * 16, 16)]
                o_v[pl.ds(i * 16, 16)] = plsc.load_gather(row_v, [idx])
                return carry3

            lax.fori_loop(0, BH // 16, gather_step, 0)
            pltpu.sync_copy(o_v, out_hbm.at[k, pl.ds(b0, BH)])
            return carry2

        lax.fori_loop(0, 2, half_step, 0)
        return carry

    lax.fori_loop(0, RPW, row_step, 0)


@jax.jit
def _sc_rowsweep(tab3, xt):
    mesh = plsc.VectorSubcoreMesh(core_axis_name="c", subcore_axis_name="s")
    return pl.kernel(
        _rowsweep_body,
        out_type=jax.ShapeDtypeStruct((K, B), jnp.float32),
        mesh=mesh,
        scratch_types=[
            pltpu.VMEM((V,), jnp.float32),
            pltpu.VMEM((BH,), jnp.int32),
            pltpu.VMEM((BH,), jnp.float32),
        ],
        compiler_params=pltpu.CompilerParams(use_tc_tiling_on_sc=False,
                                             needs_layout_passes=False),
    )(tab3, xt)


def _dense_body(ft_ref, lw_ref, lb_ref, w1_ref, b1_ref, w2_ref, b2_ref,
                out_ref):
    h = lax.dot_general(ft_ref[...], lw_ref[...], (((0,), (0,)), ((), ())),
                        preferred_element_type=jnp.float32) + lb_ref[...]
    h = jnp.maximum(jnp.dot(h, w1_ref[...],
                            preferred_element_type=jnp.float32) + b1_ref[...],
                    0.0)
    out_ref[...] = jnp.dot(h, w2_ref[...],
                           preferred_element_type=jnp.float32) + b2_ref[...]


BLK = 2048


def _tc_dense(featT, lin_w, lin_b, w1, b1, w2, b2):
    grid = (B // BLK,)
    return pl.pallas_call(
        _dense_body,
        grid=grid,
        in_specs=[
            pl.BlockSpec((K, BLK), lambda i: (0, i)),
            pl.BlockSpec((K, ADAPT), lambda i: (0, 0)),
            pl.BlockSpec((1, ADAPT), lambda i: (0, 0)),
            pl.BlockSpec((ADAPT, ADAPT), lambda i: (0, 0)),
            pl.BlockSpec((1, ADAPT), lambda i: (0, 0)),
            pl.BlockSpec((ADAPT, 1), lambda i: (0, 0)),
            pl.BlockSpec((1, 1), lambda i: (0, 0)),
        ],
        out_specs=pl.BlockSpec((BLK, 1), lambda i: (i, 0)),
        out_shape=jax.ShapeDtypeStruct((B, 1), jnp.float32),
    )(featT, lin_w, lin_b, w1, b1, w2, b2)


def kernel(x, emb_tables, lin_w, lin_b, w1, b1, w2, b2):
    tab3 = jnp.transpose(emb_tables, (0, 2, 1))
    xt = x.T
    featT = _sc_rowsweep(tab3, xt)
    return _tc_dense(featT, lin_w, lin_b.reshape(1, ADAPT), w1,
                     b1.reshape(1, ADAPT), w2, b2.reshape(1, 1))


# speedup vs baseline: 39.2242x; 2.1403x over previous
import functools

import jax
import jax.numpy as jnp
from jax import lax
from jax.experimental import pallas as pl
from jax.experimental.pallas import tpu as pltpu
from jax.experimental.pallas import tpu_sc as plsc

B = 16384
F = 26
V = 100000
D = 16
ADAPT = 128

NC = 2
NS = 16
NW = NC * NS
K = F * D
RPW = K // NW
BH = B // 2


def _rowsweep_body(tab_hbm, xt_hbm, out_hbm, row_v, x_v, o_v):
    wid = lax.axis_index("s") * NC + lax.axis_index("c")

    def row_step(j, carry):
        k = wid * RPW + j
        f = k // D
        d = k % D
        pltpu.sync_copy(tab_hbm.at[f, d], row_v)

        def half_step(h, carry2):
            b0 = h * BH
            pltpu.sync_copy(xt_hbm.at[f, pl.ds(b0, BH)], x_v)

            def gather_step(i, carry3):
                idx = x_v[pl.ds(i * 16, 16)]
                o_v[pl.ds(i * 16, 16)] = plsc.load_gather(row_v, [idx])
                return carry3

            lax.fori_loop(0, BH // 16, gather_step, 0)
            pltpu.sync_copy(o_v, out_hbm.at[k, pl.ds(b0, BH)])
            return carry2

        lax.fori_loop(0, 2, half_step, 0)
        return carry

    lax.fori_loop(0, RPW, row_step, 0)


@jax.jit
def _sc_rowsweep(tab3, xt):
    mesh = plsc.VectorSubcoreMesh(core_axis_name="c", subcore_axis_name="s")
    return pl.kernel(
        _rowsweep_body,
        out_type=jax.ShapeDtypeStruct((K, B), jnp.float32),
        mesh=mesh,
        scratch_types=[
            pltpu.VMEM((V,), jnp.float32),
            pltpu.VMEM((BH,), jnp.int32),
            pltpu.VMEM((BH,), jnp.float32),
        ],
        compiler_params=pltpu.CompilerParams(use_tc_tiling_on_sc=True,
                                             needs_layout_passes=False),
    )(tab3, xt)


def _dense_body(ft_ref, lw_ref, lb_ref, w1_ref, b1_ref, w2_ref, b2_ref,
                out_ref):
    h = lax.dot_general(ft_ref[...], lw_ref[...], (((0,), (0,)), ((), ())),
                        preferred_element_type=jnp.float32) + lb_ref[...]
    h = jnp.maximum(jnp.dot(h, w1_ref[...],
                            preferred_element_type=jnp.float32) + b1_ref[...],
                    0.0)
    out_ref[...] = jnp.dot(h, w2_ref[...],
                           preferred_element_type=jnp.float32) + b2_ref[...]


BLK = 2048


def _tc_dense(featT, lin_w, lin_b, w1, b1, w2, b2):
    grid = (B // BLK,)
    return pl.pallas_call(
        _dense_body,
        grid=grid,
        in_specs=[
            pl.BlockSpec((K, BLK), lambda i: (0, i)),
            pl.BlockSpec((K, ADAPT), lambda i: (0, 0)),
            pl.BlockSpec((1, ADAPT), lambda i: (0, 0)),
            pl.BlockSpec((ADAPT, ADAPT), lambda i: (0, 0)),
            pl.BlockSpec((1, ADAPT), lambda i: (0, 0)),
            pl.BlockSpec((ADAPT, 1), lambda i: (0, 0)),
            pl.BlockSpec((1, 1), lambda i: (0, 0)),
        ],
        out_specs=pl.BlockSpec((BLK, 1), lambda i: (i, 0)),
        out_shape=jax.ShapeDtypeStruct((B, 1), jnp.float32),
    )(featT, lin_w, lin_b, w1, b1, w2, b2)


def kernel(x, emb_tables, lin_w, lin_b, w1, b1, w2, b2):
    tab3 = jnp.transpose(emb_tables, (0, 2, 1))
    xt = x.T
    featT = _sc_rowsweep(tab3, xt)
    return _tc_dense(featT, lin_w, lin_b.reshape(1, ADAPT), w1,
                     b1.reshape(1, ADAPT), w2, b2.reshape(1, 1))


# speedup vs baseline: 42.5283x; 1.0842x over previous
import functools

import jax
import jax.numpy as jnp
from jax import lax
from jax.experimental import pallas as pl
from jax.experimental.pallas import tpu as pltpu
from jax.experimental.pallas import tpu_sc as plsc

B = 16384
F = 26
V = 100000
D = 16
ADAPT = 128

NC = 2
NS = 16
NW = NC * NS
K = F * D
RPW = K // NW
BH = B // 2


UNROLL = 4


def _rowsweep_body(tab_hbm, xt_hbm, out_hbm, row_v, x_v, o_v):
    wid = lax.axis_index("s") * NC + lax.axis_index("c")
    k0 = wid * RPW

    def row_step(j, carry):
        k = k0 + j
        f = k // D
        d = k % D

        @pl.when(jnp.logical_or(j == 0, f != (k - 1) // D))
        def _():
            pltpu.sync_copy(xt_hbm.at[f], x_v)

        pltpu.sync_copy(tab_hbm.at[f, d], row_v)

        def half_step(h):
            base = h * BH

            def body(i, c):
                for u in range(UNROLL):
                    off = (i * UNROLL + u) * 16
                    idx = x_v[pl.ds(base + off, 16)]
                    o_v[pl.ds(off, 16)] = plsc.load_gather(row_v, [idx])
                return c

            lax.fori_loop(0, BH // (16 * UNROLL), body, 0)
            pltpu.sync_copy(o_v, out_hbm.at[k, pl.ds(base, BH)])

        half_step(0)
        half_step(1)
        return carry

    lax.fori_loop(0, RPW, row_step, 0)


@jax.jit
def _sc_rowsweep(tab3, xt):
    mesh = plsc.VectorSubcoreMesh(core_axis_name="c", subcore_axis_name="s")
    return pl.kernel(
        _rowsweep_body,
        out_type=jax.ShapeDtypeStruct((K, B), jnp.float32),
        mesh=mesh,
        scratch_types=[
            pltpu.VMEM((V,), jnp.float32),
            pltpu.VMEM((B,), jnp.int32),
            pltpu.VMEM((BH,), jnp.float32),
        ],
        compiler_params=pltpu.CompilerParams(use_tc_tiling_on_sc=True,
                                             needs_layout_passes=False),
    )(tab3, xt)


def _dense_body(ft_ref, lw_ref, lb_ref, w1_ref, b1_ref, w2_ref, b2_ref,
                out_ref):
    h = lax.dot_general(ft_ref[...], lw_ref[...], (((0,), (0,)), ((), ())),
                        preferred_element_type=jnp.float32) + lb_ref[...]
    h = jnp.maximum(jnp.dot(h, w1_ref[...],
                            preferred_element_type=jnp.float32) + b1_ref[...],
                    0.0)
    out_ref[...] = jnp.dot(h, w2_ref[...],
                           preferred_element_type=jnp.float32) + b2_ref[...]


BLK = 2048


def _tc_dense(featT, lin_w, lin_b, w1, b1, w2, b2):
    grid = (B // BLK,)
    return pl.pallas_call(
        _dense_body,
        grid=grid,
        in_specs=[
            pl.BlockSpec((K, BLK), lambda i: (0, i)),
            pl.BlockSpec((K, ADAPT), lambda i: (0, 0)),
            pl.BlockSpec((1, ADAPT), lambda i: (0, 0)),
            pl.BlockSpec((ADAPT, ADAPT), lambda i: (0, 0)),
            pl.BlockSpec((1, ADAPT), lambda i: (0, 0)),
            pl.BlockSpec((ADAPT, 1), lambda i: (0, 0)),
            pl.BlockSpec((1, 1), lambda i: (0, 0)),
        ],
        out_specs=pl.BlockSpec((BLK, 1), lambda i: (i, 0)),
        out_shape=jax.ShapeDtypeStruct((B, 1), jnp.float32),
    )(featT, lin_w, lin_b, w1, b1, w2, b2)


def kernel(x, emb_tables, lin_w, lin_b, w1, b1, w2, b2):
    tab3 = jnp.transpose(emb_tables, (0, 2, 1))
    xt = x.T
    featT = _sc_rowsweep(tab3, xt)
    return _tc_dense(featT, lin_w, lin_b.reshape(1, ADAPT), w1,
                     b1.reshape(1, ADAPT), w2, b2.reshape(1, 1))


# speedup vs baseline: 50.4034x; 1.1852x over previous
import functools

import jax
import jax.numpy as jnp
from jax import lax
from jax.experimental import pallas as pl
from jax.experimental.pallas import tpu as pltpu
from jax.experimental.pallas import tpu_sc as plsc

B = 16384
F = 26
V = 100000
D = 16
ADAPT = 128

NC = 2
NS = 16
NW = NC * NS
K = F * D
RPW = K // NW
BH = B // 2


UNROLL = 8


def _rowsweep_body(tab_hbm, xt_hbm, out_hbm, row_v, x_v, o_v):
    wid = lax.axis_index("s") * NC + lax.axis_index("c")
    k0 = wid * RPW

    def row_step(j, carry):
        k = k0 + j
        f = k // D
        d = k % D

        @pl.when(jnp.logical_or(j == 0, f != (k - 1) // D))
        def _():
            pltpu.sync_copy(xt_hbm.at[f], x_v)

        pltpu.sync_copy(tab_hbm.at[f, d], row_v)

        def half_step(h):
            base = h * BH

            def body(i, c):
                for u in range(UNROLL):
                    off = (i * UNROLL + u) * 16
                    idx = x_v[pl.ds(base + off, 16)]
                    o_v[pl.ds(off, 16)] = plsc.load_gather(row_v, [idx])
                return c

            lax.fori_loop(0, BH // (16 * UNROLL), body, 0)
            pltpu.sync_copy(o_v, out_hbm.at[k, pl.ds(base, BH)])

        half_step(0)
        half_step(1)
        return carry

    lax.fori_loop(0, RPW, row_step, 0)


@jax.jit
def _sc_rowsweep(tab3, xt):
    mesh = plsc.VectorSubcoreMesh(core_axis_name="c", subcore_axis_name="s")
    return pl.kernel(
        _rowsweep_body,
        out_type=jax.ShapeDtypeStruct((K, B), jnp.float32),
        mesh=mesh,
        scratch_types=[
            pltpu.VMEM((V,), jnp.float32),
            pltpu.VMEM((B,), jnp.int32),
            pltpu.VMEM((BH,), jnp.float32),
        ],
        compiler_params=pltpu.CompilerParams(use_tc_tiling_on_sc=True,
                                             needs_layout_passes=False),
    )(tab3, xt)


def _dense_body(ft_ref, lw_ref, lb_ref, w1_ref, b1_ref, w2_ref, b2_ref,
                out_ref):
    h = lax.dot_general(ft_ref[...], lw_ref[...], (((0,), (0,)), ((), ())),
                        preferred_element_type=jnp.float32) + lb_ref[...]
    h = jnp.maximum(jnp.dot(h, w1_ref[...],
                            preferred_element_type=jnp.float32) + b1_ref[...],
                    0.0)
    out_ref[...] = jnp.dot(h, w2_ref[...],
                           preferred_element_type=jnp.float32) + b2_ref[...]


BLK = 2048


def _tc_dense(featT, lin_w, lin_b, w1, b1, w2, b2):
    grid = (B // BLK,)
    return pl.pallas_call(
        _dense_body,
        grid=grid,
        in_specs=[
            pl.BlockSpec((K, BLK), lambda i: (0, i)),
            pl.BlockSpec((K, ADAPT), lambda i: (0, 0)),
            pl.BlockSpec((1, ADAPT), lambda i: (0, 0)),
            pl.BlockSpec((ADAPT, ADAPT), lambda i: (0, 0)),
            pl.BlockSpec((1, ADAPT), lambda i: (0, 0)),
            pl.BlockSpec((ADAPT, 1), lambda i: (0, 0)),
            pl.BlockSpec((1, 1), lambda i: (0, 0)),
        ],
        out_specs=pl.BlockSpec((BLK, 1), lambda i: (i, 0)),
        out_shape=jax.ShapeDtypeStruct((B, 1), jnp.float32),
    )(featT, lin_w, lin_b, w1, b1, w2, b2)


def kernel(x, emb_tables, lin_w, lin_b, w1, b1, w2, b2):
    tab3 = jnp.transpose(emb_tables, (0, 2, 1))
    xt = x.T
    featT = _sc_rowsweep(tab3, xt)
    return _tc_dense(featT, lin_w, lin_b.reshape(1, ADAPT), w1,
                     b1.reshape(1, ADAPT), w2, b2.reshape(1, 1))
